# baseline (speedup 1.0000x reference)
import functools

import jax
import jax.numpy as jnp
from jax import lax
from jax.experimental import pallas as pl
from jax.experimental.pallas import tpu as pltpu
from jax.experimental.pallas import tpu_sc as plsc

LMAX = 2
L2 = (LMAX + 1) ** 2
C = 32
H = 64
EC = 64
N = 10000
E = 160000
F = L2 * C

_DEG = [(l * l * C, (l + 1) * (l + 1) * C) for l in range(LMAX + 1)]

BN = 1000
BE = 640
CHUNK = 128
NCHUNKS = E // CHUNK
NW = 32
HALF = F // 2
EPS = 1e-6


def _rms_norm_cols(x, gamma_row):
    outs = []
    for s, e in _DEG:
        xl = x[:, s:e]
        ms = jnp.mean(xl * xl, axis=1, keepdims=True)
        outs.append(xl * lax.rsqrt(ms + EPS))
    return jnp.concatenate(outs, axis=1) * gamma_row


def _k1_body(nf_ref, g_ref, out_ref):
    out_ref[...] = _rms_norm_cols(nf_ref[...], g_ref[...]).astype(jnp.bfloat16)


def _k1(nf2, gamma_row):
    return pl.pallas_call(
        _k1_body,
        grid=(N // BN,),
        in_specs=[
            pl.BlockSpec((BN, F), lambda i: (i, 0)),
            pl.BlockSpec((1, F), lambda i: (0, 0)),
        ],
        out_specs=pl.BlockSpec((BN, F), lambda i: (i, 0)),
        out_shape=jax.ShapeDtypeStruct((N, F), jnp.bfloat16),
    )(nf2, gamma_row)


def _k2_body(table, snd, rcv, out_s, out_r, idx_s, idx_r, buf_s, buf_r,
             sem_s, sem_r):
    wid = lax.axis_index("s") * 2 + lax.axis_index("c")

    def step(k, carry):
        cid = wid + NW * k

        @pl.when(cid < NCHUNKS)
        def _():
            base = cid * CHUNK
            pltpu.sync_copy(snd.at[pl.ds(base, CHUNK)], idx_s)
            pltpu.sync_copy(rcv.at[pl.ds(base, CHUNK)], idx_r)
            a = pltpu.async_copy(table.at[idx_s], buf_s, sem_s)
            b = pltpu.async_copy(table.at[idx_r], buf_r, sem_r)
            a.wait()
            b.wait()
            pltpu.sync_copy(buf_s, out_s.at[pl.ds(base, CHUNK)])
            pltpu.sync_copy(buf_r, out_r.at[pl.ds(base, CHUNK)])

        return carry

    lax.fori_loop(0, (NCHUNKS + NW - 1) // NW, step, 0)


def _k2(xhat, senders, receivers):
    mesh = plsc.VectorSubcoreMesh(core_axis_name="c", subcore_axis_name="s")
    fn = functools.partial(
        pl.kernel,
        mesh=mesh,
        compiler_params=pltpu.CompilerParams(use_tc_tiling_on_sc=False),
        out_type=(
            jax.ShapeDtypeStruct((E, F), jnp.bfloat16),
            jax.ShapeDtypeStruct((E, F), jnp.bfloat16),
        ),
        scratch_types=[
            pltpu.VMEM((CHUNK,), jnp.int32),
            pltpu.VMEM((CHUNK,), jnp.int32),
            pltpu.VMEM((CHUNK, F), jnp.bfloat16),
            pltpu.VMEM((CHUNK, F), jnp.bfloat16),
            pltpu.SemaphoreType.DMA,
            pltpu.SemaphoreType.DMA,
        ],
    )(_k2_body)
    return fn(xhat, senders, receivers)


def _k3_body(xs_ref, xr_ref, wigT_ref, ee_ref, envT_ref,
             we1_ref, be1_ref, we2_ref, be2_ref,
             wtop_ref, wbot_ref, wout_ref, out_ref):
    mm = functools.partial(lax.dot_general,
                           preferred_element_type=jnp.float32)
    c_maj_min = (((0,), (1,)), ((), ()))
    c_maj_maj = (((0,), (0,)), ((), ()))

    xs = xs_ref[...]
    xr = xr_ref[...]
    envT = envT_ref[...]

    e = mm(we1_ref[...], ee_ref[...], c_maj_min)
    e = e + be1_ref[...]
    e = e * jax.nn.sigmoid(e)
    e = mm(we2_ref[...], e, c_maj_maj)
    e = e + be2_ref[...]
    e = e * jax.nn.sigmoid(e)

    wtop = wtop_ref[...]
    wbot = wbot_ref[...]
    m = [
        mm(wtop, xs[:, j * C:(j + 1) * C], c_maj_min)
        + mm(wbot, xr[:, j * C:(j + 1) * C], c_maj_min)
        for j in range(L2)
    ]
    t = []
    for i in range(L2):
        acc = wigT_ref[i * L2:i * L2 + 1, :] * m[0]
        for j in range(1, L2):
            acc = acc + wigT_ref[i * L2 + j:i * L2 + j + 1, :] * m[j]
        t.append(acc * e)
    wout = wout_ref[...]
    outs = []
    for i in range(L2):
        acc = wigT_ref[i:i + 1, :] * t[0]
        for j in range(1, L2):
            acc = acc + wigT_ref[j * L2 + i:j * L2 + i + 1, :] * t[j]
        outs.append(mm(acc * envT, wout, c_maj_maj))
    out_ref[...] = jnp.concatenate(outs, axis=1)


def _k3(xs, xr, wigT, ee, envT, W_e1, b_e1, W_e2, b_e2, Wtop, Wbot, W_out):
    full = lambda shape: pl.BlockSpec(shape, lambda i: (0, 0))
    return pl.pallas_call(
        _k3_body,
        grid=(E // BE,),
        in_specs=[
            pl.BlockSpec((BE, F), lambda i: (i, 0)),
            pl.BlockSpec((BE, F), lambda i: (i, 0)),
            pl.BlockSpec((L2 * L2, BE), lambda i: (0, i)),
            pl.BlockSpec((BE, EC), lambda i: (i, 0)),
            pl.BlockSpec((1, BE), lambda i: (0, i)),
            full((EC, EC)), full((EC, 1)), full((EC, H)), full((H, 1)),
            full((C, H)), full((C, H)), full((H, C)),
        ],
        out_specs=pl.BlockSpec((BE, F), lambda i: (i, 0)),
        out_shape=jax.ShapeDtypeStruct((E, F), jnp.float32),
    )(xs, xr, wigT, ee, envT, W_e1, b_e1, W_e2, b_e2, Wtop, Wbot, W_out)


def _k4_body(p, rcv, zeros, agg, idx_v, rows_v, acc_sh, sem):
    cid = lax.axis_index("c")
    sid = lax.axis_index("s")
    col = cid * HALF
    rows_per_sub = N // 16

    pltpu.sync_copy(
        zeros.at[pl.ds(sid * rows_per_sub, rows_per_sub), pl.ds(col, HALF)],
        acc_sh.at[pl.ds(sid * rows_per_sub, rows_per_sub)])
    plsc.subcore_barrier()

    def step(k, carry):
        c = sid + 16 * k

        @pl.when(c < NCHUNKS)
        def _():
            base = c * CHUNK
            pltpu.sync_copy(rcv.at[pl.ds(base, CHUNK)], idx_v)
            pltpu.sync_copy(p.at[pl.ds(base, CHUNK), pl.ds(col, HALF)],
                            rows_v)
            pltpu.sync_copy(rows_v, acc_sh.at[idx_v], add=True)

        return carry

    lax.fori_loop(0, (NCHUNKS + 15) // 16, step, 0)
    plsc.subcore_barrier()

    pltpu.sync_copy(
        acc_sh.at[pl.ds(sid * rows_per_sub, rows_per_sub)],
        agg.at[pl.ds(sid * rows_per_sub, rows_per_sub), pl.ds(col, HALF)])


def _k4(p, receivers, zeros):
    mesh = plsc.VectorSubcoreMesh(core_axis_name="c", subcore_axis_name="s")
    fn = functools.partial(
        pl.kernel,
        mesh=mesh,
        compiler_params=pltpu.CompilerParams(use_tc_tiling_on_sc=False),
        out_type=jax.ShapeDtypeStruct((N, F), jnp.float32),
        scratch_types=[
            pltpu.VMEM((CHUNK,), jnp.int32),
            pltpu.VMEM((CHUNK, HALF), jnp.float32),
            pltpu.VMEM_SHARED((N, HALF), jnp.float32),
            pltpu.SemaphoreType.DMA,
        ],
    )(_k4_body)
    return fn(p, receivers, zeros)


def _k5_body(agg_ref, nf_ref, g_ref, w1_ref, w2_ref, out_ref):
    x = agg_ref[...] + nf_ref[...]
    xh = _rms_norm_cols(x, g_ref[...])
    w1 = w1_ref[...]
    w2 = w2_ref[...]
    h = [
        jnp.dot(xh[:, j * C:(j + 1) * C], w1,
                preferred_element_type=jnp.float32)
        for j in range(L2)
    ]
    gate = jax.nn.sigmoid(h[0])
    outs = []
    for j in range(L2):
        hj = h[0] * gate if j == 0 else h[j] * gate
        outs.append(jnp.dot(hj, w2, preferred_element_type=jnp.float32))
    out_ref[...] = jnp.concatenate(outs, axis=1) + x


def _k5(agg, nf2, gamma_row, W1, W2):
    full = lambda shape: pl.BlockSpec(shape, lambda i: (0, 0))
    return pl.pallas_call(
        _k5_body,
        grid=(N // BN,),
        in_specs=[
            pl.BlockSpec((BN, F), lambda i: (i, 0)),
            pl.BlockSpec((BN, F), lambda i: (i, 0)),
            full((1, F)), full((C, H)), full((H, C)),
        ],
        out_specs=pl.BlockSpec((BN, F), lambda i: (i, 0)),
        out_shape=jax.ShapeDtypeStruct((N, F), jnp.float32),
    )(agg, nf2, gamma_row, W1, W2)


def _gamma_row(gamma):
    parts = [
        jnp.broadcast_to(gamma[l], ((l + 1) ** 2 - l * l, C))
        for l in range(LMAX + 1)
    ]
    return jnp.concatenate(parts, axis=0).reshape(1, F)


def kernel(node_feats, edge_embeds, senders, receivers, wigner_matrices,
           edge_envelope, gamma1, gamma2, W_e1, b_e1, W_e2, b_e2, W_in,
           W_out, W1, W2):
    nf2 = node_feats.reshape(N, F)
    wigT = wigner_matrices.reshape(E, L2 * L2).T
    envT = edge_envelope.reshape(1, E)
    g1 = _gamma_row(gamma1)
    g2 = _gamma_row(gamma2)
    Wtop = W_in[:C].astype(jnp.bfloat16)
    Wbot = W_in[C:].astype(jnp.bfloat16)
    zeros = jnp.zeros((N, F), jnp.float32)

    xhat = _k1(nf2, g1)
    xs, xr = _k2(xhat, senders, receivers)
    p = _k3(xs, xr, wigT, edge_embeds, envT,
            W_e1, b_e1.reshape(EC, 1), W_e2, b_e2.reshape(H, 1),
            Wtop, Wbot, W_out)
    agg = _k4(p, receivers, zeros)
    out = _k5(agg, nf2, g2, W1, W2)
    return out.reshape(N, L2, C)


# speedup vs baseline: 1.1574x; 1.1574x over previous
import functools

import jax
import jax.numpy as jnp
from jax import lax
from jax.experimental import pallas as pl
from jax.experimental.pallas import tpu as pltpu
from jax.experimental.pallas import tpu_sc as plsc

LMAX = 2
L2 = (LMAX + 1) ** 2
C = 32
H = 64
EC = 64
N = 10000
E = 160000
F = L2 * C

_DEG = [(l * l * C, (l + 1) * (l + 1) * C) for l in range(LMAX + 1)]

BN = 1000
BE = 640
CHUNK = 128
NCHUNKS = E // CHUNK
NW = 32
HALF = F // 2
EPS = 1e-6


def _rms_norm_cols(x, gamma_row):
    outs = []
    for s, e in _DEG:
        xl = x[:, s:e]
        ms = jnp.mean(xl * xl, axis=1, keepdims=True)
        outs.append(xl * lax.rsqrt(ms + EPS))
    return jnp.concatenate(outs, axis=1) * gamma_row


def _k1_body(nf_ref, g_ref, out_ref):
    out_ref[...] = _rms_norm_cols(nf_ref[...], g_ref[...])


def _k1(nf2, gamma_row):
    return pl.pallas_call(
        _k1_body,
        grid=(N // BN,),
        in_specs=[
            pl.BlockSpec((BN, F), lambda i: (i, 0)),
            pl.BlockSpec((1, F), lambda i: (0, 0)),
        ],
        out_specs=pl.BlockSpec((BN, F), lambda i: (i, 0)),
        out_shape=jax.ShapeDtypeStruct((N, F), jnp.float32),
    )(nf2, gamma_row)


def _k2_body(nchunks, table, snd, rcv, out_s, out_r, idx_s, idx_r,
             buf_s, buf_r, sem_s, sem_r):
    wid = lax.axis_index("s") * 2 + lax.axis_index("c")

    def step(k, carry):
        cid = wid + NW * k

        @pl.when(cid < nchunks)
        def _():
            base = cid * CHUNK
            pltpu.sync_copy(snd.at[pl.ds(base, CHUNK)], idx_s)
            pltpu.sync_copy(rcv.at[pl.ds(base, CHUNK)], idx_r)
            a = pltpu.async_copy(table.at[idx_s], buf_s, sem_s)
            b = pltpu.async_copy(table.at[idx_r], buf_r, sem_r)
            a.wait()
            b.wait()
            pltpu.sync_copy(buf_s, out_s.at[pl.ds(base, CHUNK)])
            pltpu.sync_copy(buf_r, out_r.at[pl.ds(base, CHUNK)])

        return carry

    lax.fori_loop(0, (nchunks + NW - 1) // NW, step, 0)


def _k2(xhat, senders, receivers, ecount):
    nchunks = ecount // CHUNK
    mesh = plsc.VectorSubcoreMesh(core_axis_name="c", subcore_axis_name="s")
    fn = functools.partial(
        pl.kernel,
        mesh=mesh,
        compiler_params=pltpu.CompilerParams(use_tc_tiling_on_sc=False),
        out_type=(
            jax.ShapeDtypeStruct((ecount, F), jnp.float32),
            jax.ShapeDtypeStruct((ecount, F), jnp.float32),
        ),
        scratch_types=[
            pltpu.VMEM((CHUNK,), jnp.int32),
            pltpu.VMEM((CHUNK,), jnp.int32),
            pltpu.VMEM((CHUNK, F), jnp.float32),
            pltpu.VMEM((CHUNK, F), jnp.float32),
            pltpu.SemaphoreType.DMA,
            pltpu.SemaphoreType.DMA,
        ],
    )(functools.partial(_k2_body, nchunks))
    return fn(xhat, senders, receivers)


def _k3_body(xs_ref, xr_ref, wigT_ref, ee_ref, envT_ref,
             we1_ref, be1_ref, we2_ref, be2_ref,
             wtop_ref, wbot_ref, wout_ref, out_ref):
    mm = functools.partial(lax.dot_general,
                           preferred_element_type=jnp.float32)
    c_maj_min = (((0,), (1,)), ((), ()))
    c_maj_maj = (((0,), (0,)), ((), ()))

    xs = xs_ref[...]
    xr = xr_ref[...]
    envT = envT_ref[...]

    e = mm(we1_ref[...], ee_ref[...], c_maj_min)
    e = e + be1_ref[...]
    e = e * jax.nn.sigmoid(e)
    e = mm(we2_ref[...], e, c_maj_maj)
    e = e + be2_ref[...]
    e = e * jax.nn.sigmoid(e)

    wtop = wtop_ref[...]
    wbot = wbot_ref[...]
    m = [
        mm(wtop, xs[:, j * C:(j + 1) * C], c_maj_min)
        + mm(wbot, xr[:, j * C:(j + 1) * C], c_maj_min)
        for j in range(L2)
    ]
    t = []
    for i in range(L2):
        acc = wigT_ref[i * L2:i * L2 + 1, :] * m[0]
        for j in range(1, L2):
            acc = acc + wigT_ref[i * L2 + j:i * L2 + j + 1, :] * m[j]
        t.append(acc * e)
    wout = wout_ref[...]
    outs = []
    for i in range(L2):
        acc = wigT_ref[i:i + 1, :] * t[0]
        for j in range(1, L2):
            acc = acc + wigT_ref[j * L2 + i:j * L2 + i + 1, :] * t[j]
        outs.append(mm(acc * envT, wout, c_maj_maj))
    out_ref[...] = jnp.concatenate(outs, axis=1)


def _k3(xs, xr, wigT, ee, envT, W_e1, b_e1, W_e2, b_e2, Wtop, Wbot, W_out):
    ecount = xs.shape[0]
    full = lambda shape: pl.BlockSpec(shape, lambda i: (0, 0))
    return pl.pallas_call(
        _k3_body,
        grid=(ecount // BE,),
        in_specs=[
            pl.BlockSpec((BE, F), lambda i: (i, 0)),
            pl.BlockSpec((BE, F), lambda i: (i, 0)),
            pl.BlockSpec((L2 * L2, BE), lambda i: (0, i)),
            pl.BlockSpec((BE, EC), lambda i: (i, 0)),
            pl.BlockSpec((1, BE), lambda i: (0, i)),
            full((EC, EC)), full((EC, 1)), full((EC, H)), full((H, 1)),
            full((C, H)), full((C, H)), full((H, C)),
        ],
        out_specs=pl.BlockSpec((BE, F), lambda i: (i, 0)),
        out_shape=jax.ShapeDtypeStruct((ecount, F), jnp.float32),
    )(xs, xr, wigT, ee, envT, W_e1, b_e1, W_e2, b_e2, Wtop, Wbot, W_out)


def _k4_body(nchunks, p, rcv, zeros, agg, idx_v, rows_v, acc_sh, sem):
    cid = lax.axis_index("c")
    sid = lax.axis_index("s")
    col = cid * HALF
    rows_per_sub = N // 16

    pltpu.sync_copy(
        zeros.at[pl.ds(sid * rows_per_sub, rows_per_sub), pl.ds(col, HALF)],
        acc_sh.at[pl.ds(sid * rows_per_sub, rows_per_sub)])
    plsc.subcore_barrier()

    def step(k, carry):
        c = sid + 16 * k

        @pl.when(c < nchunks)
        def _():
            base = c * CHUNK
            pltpu.sync_copy(rcv.at[pl.ds(base, CHUNK)], idx_v)
            pltpu.sync_copy(p.at[pl.ds(base, CHUNK), pl.ds(col, HALF)],
                            rows_v)
            pltpu.sync_copy(rows_v, acc_sh.at[idx_v], add=True)

        return carry

    lax.fori_loop(0, (nchunks + 15) // 16, step, 0)
    plsc.subcore_barrier()

    pltpu.sync_copy(
        acc_sh.at[pl.ds(sid * rows_per_sub, rows_per_sub)],
        agg.at[pl.ds(sid * rows_per_sub, rows_per_sub), pl.ds(col, HALF)])


def _k4(p, receivers, zeros):
    nchunks = p.shape[0] // CHUNK
    mesh = plsc.VectorSubcoreMesh(core_axis_name="c", subcore_axis_name="s")
    fn = functools.partial(
        pl.kernel,
        mesh=mesh,
        compiler_params=pltpu.CompilerParams(use_tc_tiling_on_sc=False),
        out_type=jax.ShapeDtypeStruct((N, F), jnp.float32),
        scratch_types=[
            pltpu.VMEM((CHUNK,), jnp.int32),
            pltpu.VMEM((CHUNK, HALF), jnp.float32),
            pltpu.VMEM_SHARED((N, HALF), jnp.float32),
            pltpu.SemaphoreType.DMA,
        ],
    )(functools.partial(_k4_body, nchunks))
    return fn(p, receivers, zeros)


def _k5_body(agga_ref, aggb_ref, nf_ref, g_ref, w1_ref, w2_ref, out_ref):
    x = agga_ref[...] + aggb_ref[...] + nf_ref[...]
    xh = _rms_norm_cols(x, g_ref[...])
    w1 = w1_ref[...]
    w2 = w2_ref[...]
    h = [
        jnp.dot(xh[:, j * C:(j + 1) * C], w1,
                preferred_element_type=jnp.float32)
        for j in range(L2)
    ]
    gate = jax.nn.sigmoid(h[0])
    outs = []
    for j in range(L2):
        hj = h[0] * gate if j == 0 else h[j] * gate
        outs.append(jnp.dot(hj, w2, preferred_element_type=jnp.float32))
    out_ref[...] = jnp.concatenate(outs, axis=1) + x


def _k5(agga, aggb, nf2, gamma_row, W1, W2):
    full = lambda shape: pl.BlockSpec(shape, lambda i: (0, 0))
    return pl.pallas_call(
        _k5_body,
        grid=(N // BN,),
        in_specs=[
            pl.BlockSpec((BN, F), lambda i: (i, 0)),
            pl.BlockSpec((BN, F), lambda i: (i, 0)),
            pl.BlockSpec((BN, F), lambda i: (i, 0)),
            full((1, F)), full((C, H)), full((H, C)),
        ],
        out_specs=pl.BlockSpec((BN, F), lambda i: (i, 0)),
        out_shape=jax.ShapeDtypeStruct((N, F), jnp.float32),
    )(agga, aggb, nf2, gamma_row, W1, W2)


def _gamma_row(gamma):
    parts = [
        jnp.broadcast_to(gamma[l], ((l + 1) ** 2 - l * l, C))
        for l in range(LMAX + 1)
    ]
    return jnp.concatenate(parts, axis=0).reshape(1, F)


def kernel(node_feats, edge_embeds, senders, receivers, wigner_matrices,
           edge_envelope, gamma1, gamma2, W_e1, b_e1, W_e2, b_e2, W_in,
           W_out, W1, W2):
    nf2 = node_feats.reshape(N, F)
    wigT = wigner_matrices.reshape(E, L2 * L2).T
    envT = edge_envelope.reshape(1, E)
    g1 = _gamma_row(gamma1)
    g2 = _gamma_row(gamma2)
    Wtop = W_in[:C]
    Wbot = W_in[C:]
    zeros = jnp.zeros((N, F), jnp.float32)

    be1c = b_e1.reshape(EC, 1)
    be2c = b_e2.reshape(H, 1)
    EH = E // 2

    xhat = _k1(nf2, g1)
    xs_a, xr_a = _k2(xhat, senders[:EH], receivers[:EH], EH)
    xs_b, xr_b = _k2(xhat, senders[EH:], receivers[EH:], EH)
    p_a = _k3(xs_a, xr_a, wigT[:, :EH], edge_embeds[:EH], envT[:, :EH],
              W_e1, be1c, W_e2, be2c, Wtop, Wbot, W_out)
    p_b = _k3(xs_b, xr_b, wigT[:, EH:], edge_embeds[EH:], envT[:, EH:],
              W_e1, be1c, W_e2, be2c, Wtop, Wbot, W_out)
    agg_a = _k4(p_a, receivers[:EH], zeros)
    agg_b = _k4(p_b, receivers[EH:], zeros)
    out = _k5(agg_a, agg_b, nf2, g2, W1, W2)
    return out.reshape(N, L2, C)


# speedup vs baseline: 1.1762x; 1.0162x over previous
import functools

import jax
import jax.numpy as jnp
from jax import lax
from jax.experimental import pallas as pl
from jax.experimental.pallas import tpu as pltpu
from jax.experimental.pallas import tpu_sc as plsc

LMAX = 2
L2 = (LMAX + 1) ** 2
C = 32
H = 64
EC = 64
N = 10000
E = 160000
F = L2 * C

_DEG = [(l * l * C, (l + 1) * (l + 1) * C) for l in range(LMAX + 1)]

BN = 1000
BE = 640
CHUNK = 128
NCHUNKS = E // CHUNK
NW = 32
HALF = F // 2
EPS = 1e-6


def _rms_norm_cols(x, gamma_row):
    outs = []
    for s, e in _DEG:
        xl = x[:, s:e]
        ms = jnp.mean(xl * xl, axis=1, keepdims=True)
        outs.append(xl * lax.rsqrt(ms + EPS))
    return jnp.concatenate(outs, axis=1) * gamma_row


def _k1_body(nf_ref, g_ref, out_ref):
    out_ref[...] = _rms_norm_cols(nf_ref[...], g_ref[...])


def _k1(nf2, gamma_row):
    return pl.pallas_call(
        _k1_body,
        grid=(N // BN,),
        in_specs=[
            pl.BlockSpec((BN, F), lambda i: (i, 0)),
            pl.BlockSpec((1, F), lambda i: (0, 0)),
        ],
        out_specs=pl.BlockSpec((BN, F), lambda i: (i, 0)),
        out_shape=jax.ShapeDtypeStruct((N, F), jnp.float32),
    )(nf2, gamma_row)


def _k2_body(nchunks, table, snd, rcv, out_s, out_r, idx_s, idx_r,
             buf_s, buf_r, sem_s, sem_r):
    wid = lax.axis_index("s") * 2 + lax.axis_index("c")

    def step(k, carry):
        cid = wid + NW * k

        @pl.when(cid < nchunks)
        def _():
            base = cid * CHUNK
            pltpu.sync_copy(snd.at[pl.ds(base, CHUNK)], idx_s)
            pltpu.sync_copy(rcv.at[pl.ds(base, CHUNK)], idx_r)
            a = pltpu.async_copy(table.at[idx_s], buf_s, sem_s)
            b = pltpu.async_copy(table.at[idx_r], buf_r, sem_r)
            a.wait()
            b.wait()
            pltpu.sync_copy(buf_s, out_s.at[pl.ds(base, CHUNK)])
            pltpu.sync_copy(buf_r, out_r.at[pl.ds(base, CHUNK)])

        return carry

    lax.fori_loop(0, (nchunks + NW - 1) // NW, step, 0)


def _k2(xhat, senders, receivers, ecount):
    nchunks = ecount // CHUNK
    mesh = plsc.VectorSubcoreMesh(core_axis_name="c", subcore_axis_name="s")
    fn = functools.partial(
        pl.kernel,
        mesh=mesh,
        compiler_params=pltpu.CompilerParams(use_tc_tiling_on_sc=False),
        out_type=(
            jax.ShapeDtypeStruct((ecount, F), jnp.float32),
            jax.ShapeDtypeStruct((ecount, F), jnp.float32),
        ),
        scratch_types=[
            pltpu.VMEM((CHUNK,), jnp.int32),
            pltpu.VMEM((CHUNK,), jnp.int32),
            pltpu.VMEM((CHUNK, F), jnp.float32),
            pltpu.VMEM((CHUNK, F), jnp.float32),
            pltpu.SemaphoreType.DMA,
            pltpu.SemaphoreType.DMA,
        ],
    )(functools.partial(_k2_body, nchunks))
    return fn(xhat, senders, receivers)


def _k3_body(xs_ref, xr_ref, wigT_ref, ee_ref, envT_ref,
             we1_ref, be1_ref, we2_ref, be2_ref,
             wtop_ref, wbot_ref, wout_ref, out_ref):
    mm = functools.partial(lax.dot_general,
                           preferred_element_type=jnp.float32)
    c_maj_min = (((0,), (1,)), ((), ()))
    c_maj_maj = (((0,), (0,)), ((), ()))

    xs = xs_ref[...]
    xr = xr_ref[...]
    envT = envT_ref[...]

    e = mm(we1_ref[...], ee_ref[...], c_maj_min)
    e = e + be1_ref[...]
    e = e * jax.nn.sigmoid(e)
    e = mm(we2_ref[...], e, c_maj_maj)
    e = e + be2_ref[...]
    e = e * jax.nn.sigmoid(e)

    wtop = wtop_ref[...]
    wbot = wbot_ref[...]
    m = [
        mm(wtop, xs[:, j * C:(j + 1) * C], c_maj_min)
        + mm(wbot, xr[:, j * C:(j + 1) * C], c_maj_min)
        for j in range(L2)
    ]
    t = []
    for i in range(L2):
        acc = wigT_ref[i * L2:i * L2 + 1, :] * m[0]
        for j in range(1, L2):
            acc = acc + wigT_ref[i * L2 + j:i * L2 + j + 1, :] * m[j]
        t.append(acc * e)
    wout = wout_ref[...]
    outs = []
    for i in range(L2):
        acc = wigT_ref[i:i + 1, :] * t[0]
        for j in range(1, L2):
            acc = acc + wigT_ref[j * L2 + i:j * L2 + i + 1, :] * t[j]
        outs.append(mm(acc * envT, wout, c_maj_maj))
    out_ref[...] = jnp.concatenate(outs, axis=1)


def _k3(xs, xr, wigT, ee, envT, W_e1, b_e1, W_e2, b_e2, Wtop, Wbot, W_out):
    ecount = xs.shape[0]
    full = lambda shape: pl.BlockSpec(shape, lambda i: (0, 0))
    return pl.pallas_call(
        _k3_body,
        grid=(ecount // BE,),
        in_specs=[
            pl.BlockSpec((BE, F), lambda i: (i, 0)),
            pl.BlockSpec((BE, F), lambda i: (i, 0)),
            pl.BlockSpec((L2 * L2, BE), lambda i: (0, i)),
            pl.BlockSpec((BE, EC), lambda i: (i, 0)),
            pl.BlockSpec((1, BE), lambda i: (0, i)),
            full((EC, EC)), full((EC, 1)), full((EC, H)), full((H, 1)),
            full((C, H)), full((C, H)), full((H, C)),
        ],
        out_specs=pl.BlockSpec((BE, F), lambda i: (i, 0)),
        out_shape=jax.ShapeDtypeStruct((ecount, F), jnp.float32),
    )(xs, xr, wigT, ee, envT, W_e1, b_e1, W_e2, b_e2, Wtop, Wbot, W_out)


def _k4_body(nchunks, p, rcv, zeros, agg, idx_v, rows_v, acc_sh, sem):
    cid = lax.axis_index("c")
    sid = lax.axis_index("s")
    col = cid * HALF
    rows_per_sub = N // 16

    pltpu.sync_copy(
        zeros.at[pl.ds(sid * rows_per_sub, rows_per_sub), pl.ds(col, HALF)],
        acc_sh.at[pl.ds(sid * rows_per_sub, rows_per_sub)])
    plsc.subcore_barrier()

    def step(k, carry):
        c = sid + 16 * k

        @pl.when(c < nchunks)
        def _():
            base = c * CHUNK
            pltpu.sync_copy(rcv.at[pl.ds(base, CHUNK)], idx_v)
            pltpu.sync_copy(p.at[pl.ds(base, CHUNK), pl.ds(col, HALF)],
                            rows_v)
            pltpu.sync_copy(rows_v, acc_sh.at[idx_v], add=True)

        return carry

    lax.fori_loop(0, (nchunks + 15) // 16, step, 0)
    plsc.subcore_barrier()

    pltpu.sync_copy(
        acc_sh.at[pl.ds(sid * rows_per_sub, rows_per_sub)],
        agg.at[pl.ds(sid * rows_per_sub, rows_per_sub), pl.ds(col, HALF)])


def _k4(p, receivers, zeros):
    nchunks = p.shape[0] // CHUNK
    mesh = plsc.VectorSubcoreMesh(core_axis_name="c", subcore_axis_name="s")
    fn = functools.partial(
        pl.kernel,
        mesh=mesh,
        compiler_params=pltpu.CompilerParams(use_tc_tiling_on_sc=False),
        out_type=jax.ShapeDtypeStruct((N, F), jnp.float32),
        scratch_types=[
            pltpu.VMEM((CHUNK,), jnp.int32),
            pltpu.VMEM((CHUNK, HALF), jnp.float32),
            pltpu.VMEM_SHARED((N, HALF), jnp.float32),
            pltpu.SemaphoreType.DMA,
        ],
    )(functools.partial(_k4_body, nchunks))
    return fn(p, receivers, zeros)


def _k5_body(agga_ref, aggb_ref, aggc_ref, aggd_ref, nf_ref, g_ref,
             w1_ref, w2_ref, out_ref):
    x = ((agga_ref[...] + aggb_ref[...]) + (aggc_ref[...] + aggd_ref[...])
         + nf_ref[...])
    xh = _rms_norm_cols(x, g_ref[...])
    w1 = w1_ref[...]
    w2 = w2_ref[...]
    h = [
        jnp.dot(xh[:, j * C:(j + 1) * C], w1,
                preferred_element_type=jnp.float32)
        for j in range(L2)
    ]
    gate = jax.nn.sigmoid(h[0])
    outs = []
    for j in range(L2):
        hj = h[0] * gate if j == 0 else h[j] * gate
        outs.append(jnp.dot(hj, w2, preferred_element_type=jnp.float32))
    out_ref[...] = jnp.concatenate(outs, axis=1) + x


def _k5(aggs, nf2, gamma_row, W1, W2):
    full = lambda shape: pl.BlockSpec(shape, lambda i: (0, 0))
    blk = lambda: pl.BlockSpec((BN, F), lambda i: (i, 0))
    return pl.pallas_call(
        _k5_body,
        grid=(N // BN,),
        in_specs=[
            blk(), blk(), blk(), blk(), blk(),
            full((1, F)), full((C, H)), full((H, C)),
        ],
        out_specs=pl.BlockSpec((BN, F), lambda i: (i, 0)),
        out_shape=jax.ShapeDtypeStruct((N, F), jnp.float32),
    )(*aggs, nf2, gamma_row, W1, W2)


def _gamma_row(gamma):
    parts = [
        jnp.broadcast_to(gamma[l], ((l + 1) ** 2 - l * l, C))
        for l in range(LMAX + 1)
    ]
    return jnp.concatenate(parts, axis=0).reshape(1, F)


def kernel(node_feats, edge_embeds, senders, receivers, wigner_matrices,
           edge_envelope, gamma1, gamma2, W_e1, b_e1, W_e2, b_e2, W_in,
           W_out, W1, W2):
    nf2 = node_feats.reshape(N, F)
    wigT = wigner_matrices.reshape(E, L2 * L2).T
    envT = edge_envelope.reshape(1, E)
    g1 = _gamma_row(gamma1)
    g2 = _gamma_row(gamma2)
    Wtop = W_in[:C]
    Wbot = W_in[C:]
    zeros = jnp.zeros((N, F), jnp.float32)

    be1c = b_e1.reshape(EC, 1)
    be2c = b_e2.reshape(H, 1)
    sizes = (40960, 40960, 40960, 37120)
    offs = (0, 40960, 81920, 122880)

    xhat = _k1(nf2, g1)
    gathered = [
        _k2(xhat, senders[o:o + n], receivers[o:o + n], n)
        for o, n in zip(offs, sizes)
    ]
    ps = [
        _k3(xs, xr, wigT[:, o:o + n], edge_embeds[o:o + n],
            envT[:, o:o + n], W_e1, be1c, W_e2, be2c, Wtop, Wbot, W_out)
        for (xs, xr), o, n in zip(gathered, offs, sizes)
    ]
    aggs = [
        _k4(pp, receivers[o:o + n], zeros)
        for pp, o, n in zip(ps, offs, sizes)
    ]
    out = _k5(aggs, nf2, g2, W1, W2)
    return out.reshape(N, L2, C)


# speedup vs baseline: 1.1948x; 1.0159x over previous
import functools

import jax
import jax.numpy as jnp
from jax import lax
from jax.experimental import pallas as pl
from jax.experimental.pallas import tpu as pltpu
from jax.experimental.pallas import tpu_sc as plsc

LMAX = 2
L2 = (LMAX + 1) ** 2
C = 32
H = 64
EC = 64
N = 10000
E = 160000
F = L2 * C

_DEG = [(l * l * C, (l + 1) * (l + 1) * C) for l in range(LMAX + 1)]

BN = 1000
BE = 640
CHUNK = 128
NCHUNKS = E // CHUNK
NW = 32
HALF = F // 2
EPS = 1e-6


def _rms_norm_cols(x, gamma_row):
    outs = []
    for s, e in _DEG:
        xl = x[:, s:e]
        ms = jnp.mean(xl * xl, axis=1, keepdims=True)
        outs.append(xl * lax.rsqrt(ms + EPS))
    return jnp.concatenate(outs, axis=1) * gamma_row


def _k1_body(nf_ref, g_ref, out_ref):
    out_ref[...] = _rms_norm_cols(nf_ref[...], g_ref[...])


def _k1(nf2, gamma_row):
    return pl.pallas_call(
        _k1_body,
        grid=(N // BN,),
        in_specs=[
            pl.BlockSpec((BN, F), lambda i: (i, 0)),
            pl.BlockSpec((1, F), lambda i: (0, 0)),
        ],
        out_specs=pl.BlockSpec((BN, F), lambda i: (i, 0)),
        out_shape=jax.ShapeDtypeStruct((N, F), jnp.float32),
    )(nf2, gamma_row)


def _k2_body(nchunks, table, snd, rcv, out_s, out_r, idx_s, idx_r,
             buf_s, buf_r, sem_s, sem_r):
    wid = lax.axis_index("s") * 2 + lax.axis_index("c")

    def step(k, carry):
        cid = wid + NW * k

        @pl.when(cid < nchunks)
        def _():
            base = cid * CHUNK
            pltpu.sync_copy(snd.at[pl.ds(base, CHUNK)], idx_s)
            pltpu.sync_copy(rcv.at[pl.ds(base, CHUNK)], idx_r)
            a = pltpu.async_copy(table.at[idx_s], buf_s, sem_s)
            b = pltpu.async_copy(table.at[idx_r], buf_r, sem_r)
            a.wait()
            b.wait()
            pltpu.sync_copy(buf_s, out_s.at[pl.ds(base, CHUNK)])
            pltpu.sync_copy(buf_r, out_r.at[pl.ds(base, CHUNK)])

        return carry

    lax.fori_loop(0, (nchunks + NW - 1) // NW, step, 0)


def _k2(xhat, senders, receivers, ecount):
    nchunks = ecount // CHUNK
    mesh = plsc.VectorSubcoreMesh(core_axis_name="c", subcore_axis_name="s")
    fn = functools.partial(
        pl.kernel,
        mesh=mesh,
        compiler_params=pltpu.CompilerParams(use_tc_tiling_on_sc=False),
        out_type=(
            jax.ShapeDtypeStruct((ecount, F), jnp.float32),
            jax.ShapeDtypeStruct((ecount, F), jnp.float32),
        ),
        scratch_types=[
            pltpu.VMEM((CHUNK,), jnp.int32),
            pltpu.VMEM((CHUNK,), jnp.int32),
            pltpu.VMEM((CHUNK, F), jnp.float32),
            pltpu.VMEM((CHUNK, F), jnp.float32),
            pltpu.SemaphoreType.DMA,
            pltpu.SemaphoreType.DMA,
        ],
    )(functools.partial(_k2_body, nchunks))
    return fn(xhat, senders, receivers)


def _k3_body(xs_ref, xr_ref, wigT_ref, ee_ref, envT_ref,
             we1_ref, be1_ref, we2_ref, be2_ref,
             wtop_ref, wbot_ref, wout_ref, out_ref):
    mm = functools.partial(lax.dot_general,
                           preferred_element_type=jnp.float32)
    c_maj_min = (((0,), (1,)), ((), ()))
    c_maj_maj = (((0,), (0,)), ((), ()))

    xs = xs_ref[...]
    xr = xr_ref[...]
    envT = envT_ref[...]

    e = mm(we1_ref[...], ee_ref[...], c_maj_min)
    e = e + be1_ref[...]
    e = e * jax.nn.sigmoid(e)
    e = mm(we2_ref[...], e, c_maj_maj)
    e = e + be2_ref[...]
    e = e * jax.nn.sigmoid(e)

    wtop = wtop_ref[...]
    wbot = wbot_ref[...]
    m = [
        mm(wtop, xs[:, j * C:(j + 1) * C], c_maj_min)
        + mm(wbot, xr[:, j * C:(j + 1) * C], c_maj_min)
        for j in range(L2)
    ]
    t = []
    for i in range(L2):
        acc = wigT_ref[i * L2:i * L2 + 1, :] * m[0]
        for j in range(1, L2):
            acc = acc + wigT_ref[i * L2 + j:i * L2 + j + 1, :] * m[j]
        t.append(acc * e)
    wout = wout_ref[...]
    outs = []
    for i in range(L2):
        acc = wigT_ref[i:i + 1, :] * t[0]
        for j in range(1, L2):
            acc = acc + wigT_ref[j * L2 + i:j * L2 + i + 1, :] * t[j]
        outs.append(mm(acc * envT, wout, c_maj_maj))
    out_ref[...] = jnp.concatenate(outs, axis=1)


def _k3(xs, xr, wigT, ee, envT, W_e1, b_e1, W_e2, b_e2, Wtop, Wbot, W_out):
    ecount = xs.shape[0]
    full = lambda shape: pl.BlockSpec(shape, lambda i: (0, 0))
    return pl.pallas_call(
        _k3_body,
        grid=(ecount // BE,),
        in_specs=[
            pl.BlockSpec((BE, F), lambda i: (i, 0)),
            pl.BlockSpec((BE, F), lambda i: (i, 0)),
            pl.BlockSpec((L2 * L2, BE), lambda i: (0, i)),
            pl.BlockSpec((BE, EC), lambda i: (i, 0)),
            pl.BlockSpec((1, BE), lambda i: (0, i)),
            full((EC, EC)), full((EC, 1)), full((EC, H)), full((H, 1)),
            full((C, H)), full((C, H)), full((H, C)),
        ],
        out_specs=pl.BlockSpec((BE, F), lambda i: (i, 0)),
        out_shape=jax.ShapeDtypeStruct((ecount, F), jnp.float32),
    )(xs, xr, wigT, ee, envT, W_e1, b_e1, W_e2, b_e2, Wtop, Wbot, W_out)


def _k4_body(nchunks, p, rcv, zeros, agg, idx_v, rows_v, acc_sh, sem):
    cid = lax.axis_index("c")
    sid = lax.axis_index("s")
    col = cid * HALF
    rows_per_sub = N // 16

    pltpu.sync_copy(
        zeros.at[pl.ds(sid * rows_per_sub, rows_per_sub), pl.ds(col, HALF)],
        acc_sh.at[pl.ds(sid * rows_per_sub, rows_per_sub)])
    plsc.subcore_barrier()

    def step(k, carry):
        c = sid + 16 * k

        @pl.when(c < nchunks)
        def _():
            base = c * CHUNK
            pltpu.sync_copy(rcv.at[pl.ds(base, CHUNK)], idx_v)
            pltpu.sync_copy(p.at[pl.ds(base, CHUNK), pl.ds(col, HALF)],
                            rows_v)
            pltpu.sync_copy(rows_v, acc_sh.at[idx_v], add=True)

        return carry

    lax.fori_loop(0, (nchunks + 15) // 16, step, 0)
    plsc.subcore_barrier()

    pltpu.sync_copy(
        acc_sh.at[pl.ds(sid * rows_per_sub, rows_per_sub)],
        agg.at[pl.ds(sid * rows_per_sub, rows_per_sub), pl.ds(col, HALF)])


def _k4(p, receivers, zeros):
    nchunks = p.shape[0] // CHUNK
    mesh = plsc.VectorSubcoreMesh(core_axis_name="c", subcore_axis_name="s")
    fn = functools.partial(
        pl.kernel,
        mesh=mesh,
        compiler_params=pltpu.CompilerParams(use_tc_tiling_on_sc=False),
        out_type=jax.ShapeDtypeStruct((N, F), jnp.float32),
        scratch_types=[
            pltpu.VMEM((CHUNK,), jnp.int32),
            pltpu.VMEM((CHUNK, HALF), jnp.float32),
            pltpu.VMEM_SHARED((N, HALF), jnp.float32),
            pltpu.SemaphoreType.DMA,
        ],
    )(functools.partial(_k4_body, nchunks))
    return fn(p, receivers, zeros)


def _k5_body(agga_ref, aggb_ref, aggc_ref, aggd_ref, nf_ref, g_ref,
             w1_ref, w2_ref, out_ref):
    x = ((agga_ref[...] + aggb_ref[...]) + (aggc_ref[...] + aggd_ref[...])
         + nf_ref[...])
    xh = _rms_norm_cols(x, g_ref[...])
    mm = functools.partial(lax.dot_general,
                           preferred_element_type=jnp.float32)
    c_maj_min = (((0,), (1,)), ((), ()))
    c_maj_maj = (((0,), (0,)), ((), ()))
    w1 = w1_ref[...]
    w2 = w2_ref[...]
    h = [mm(w1, xh[:, j * C:(j + 1) * C], c_maj_min) for j in range(L2)]
    gate = jax.nn.sigmoid(h[0])
    outs = [mm(h[j] * gate, w2, c_maj_maj) for j in range(L2)]
    out_ref[...] = jnp.concatenate(outs, axis=1) + x


def _k5(aggs, nf2, gamma_row, W1, W2):
    full = lambda shape: pl.BlockSpec(shape, lambda i: (0, 0))
    blk = lambda: pl.BlockSpec((BN, F), lambda i: (i, 0))
    return pl.pallas_call(
        _k5_body,
        grid=(N // BN,),
        in_specs=[
            blk(), blk(), blk(), blk(), blk(),
            full((1, F)), full((C, H)), full((H, C)),
        ],
        out_specs=pl.BlockSpec((BN, F), lambda i: (i, 0)),
        out_shape=jax.ShapeDtypeStruct((N, F), jnp.float32),
    )(*aggs, nf2, gamma_row, W1, W2)


def _gamma_row(gamma):
    parts = [
        jnp.broadcast_to(gamma[l], ((l + 1) ** 2 - l * l, C))
        for l in range(LMAX + 1)
    ]
    return jnp.concatenate(parts, axis=0).reshape(1, F)


def kernel(node_feats, edge_embeds, senders, receivers, wigner_matrices,
           edge_envelope, gamma1, gamma2, W_e1, b_e1, W_e2, b_e2, W_in,
           W_out, W1, W2):
    nf2 = node_feats.reshape(N, F)
    wigT = wigner_matrices.reshape(E, L2 * L2).T
    envT = edge_envelope.reshape(1, E)
    g1 = _gamma_row(gamma1)
    g2 = _gamma_row(gamma2)
    Wtop = W_in[:C]
    Wbot = W_in[C:]
    zeros = jnp.zeros((N, F), jnp.float32)

    be1c = b_e1.reshape(EC, 1)
    be2c = b_e2.reshape(H, 1)
    sizes = (40960, 40960, 40960, 37120)
    offs = (0, 40960, 81920, 122880)

    xhat = _k1(nf2, g1)
    gathered = [
        _k2(xhat, senders[o:o + n], receivers[o:o + n], n)
        for o, n in zip(offs, sizes)
    ]
    ps = [
        _k3(xs, xr, wigT[:, o:o + n], edge_embeds[o:o + n],
            envT[:, o:o + n], W_e1, be1c, W_e2, be2c, Wtop, Wbot, W_out)
        for (xs, xr), o, n in zip(gathered, offs, sizes)
    ]
    aggs = [
        _k4(pp, receivers[o:o + n], zeros)
        for pp, o, n in zip(ps, offs, sizes)
    ]
    out = _k5(aggs, nf2, g2, W1, W2)
    return out.reshape(N, L2, C)
